# baseline (speedup 1.0000x reference)
import jax
import jax.numpy as jnp
from jax.experimental import pallas as pl
from jax.experimental.pallas import tpu as pltpu

_S, _D, _H, _HD, _E, _K, _DFF = 2048, 1024, 16, 64, 8, 2, 4096
_BM = 256
_BQ = 256


def _ln_rows(x, scale, bias):
    m = jnp.mean(x, axis=-1, keepdims=True)
    xc = x - m
    v = jnp.mean(xc * xc, axis=-1, keepdims=True)
    return xc * jax.lax.rsqrt(v + 1e-5) * scale + bias


def _mm(a, b):
    return jax.lax.dot_general(
        a.astype(jnp.bfloat16), b.astype(jnp.bfloat16),
        (((1,), (0,)), ((), ())), preferred_element_type=jnp.float32)


def _mmT(a, b):
    return jax.lax.dot_general(
        a.astype(jnp.bfloat16), b.astype(jnp.bfloat16),
        (((1,), (1,)), ((), ())), preferred_element_type=jnp.float32)


def _qkv_kernel(x_ref, w_ref, b_ref, s_ref, t_ref, o_ref):
    xn = _ln_rows(x_ref[...], s_ref[...], t_ref[...])
    o_ref[...] = _mm(xn, w_ref[...]) + b_ref[...]


def _attn_kernel(q_ref, k_ref, v_ref, m_ref, w_ref, o_ref):
    s = _mmT(q_ref[0], k_ref[0]) * (_HD ** -0.5)
    s = jnp.where(m_ref[...] == 0.0, -1e30, s)
    mx = jnp.max(s, axis=-1, keepdims=True)
    p = jnp.exp(s - mx)
    p = p / jnp.sum(p, axis=-1, keepdims=True)
    w_ref[...] = p.reshape(1, 1, _BQ, _S)
    o_ref[...] = _mm(p, v_ref[0]).reshape(1, _BQ, _HD)


def _proj_kernel(a_ref, w_ref, b_ref, x_ref, o_ref):
    o_ref[...] = x_ref[...] + _mm(a_ref[...], w_ref[...]) + b_ref[...]


def _router_kernel(y_ref, s_ref, b_ref, wr_ref, br_ref, n_ref, p_ref, g_ref,
                   q_ref):
    yn = _ln_rows(y_ref[...], s_ref[...], b_ref[...])
    n_ref[...] = yn
    logits = _mm(yn, wr_ref[...]) + br_ref[...]
    iota = jax.lax.broadcasted_iota(jnp.int32, (_S, _E), 1)
    m1 = jnp.max(logits, axis=-1, keepdims=True)
    i1 = jnp.min(jnp.where(logits == m1, iota, _E), axis=-1, keepdims=True)
    l2 = jnp.where(iota == i1, -jnp.inf, logits)
    m2 = jnp.max(l2, axis=-1, keepdims=True)
    i2 = jnp.min(jnp.where(l2 == m2, iota, _E), axis=-1, keepdims=True)
    p1 = 1.0 / (1.0 + jnp.exp(m2 - m1))
    p2 = 1.0 - p1
    p_ref[...] = jnp.where(iota == 0, p1, jnp.where(iota == 1, p2, 0.0))
    g = (jnp.where(iota == i1, p1, 0.0)
         + jnp.where(iota == i2, p2, 0.0))
    g_ref[...] = g
    c = (g > 0.0).astype(jnp.bfloat16)
    ti = jax.lax.broadcasted_iota(jnp.int32, (_S, _S), 0)
    tj = jax.lax.broadcasted_iota(jnp.int32, (_S, _S), 1)
    lstrict = (tj < ti).astype(jnp.bfloat16)
    q_ref[...] = jax.lax.dot_general(
        lstrict, c, (((1,), (0,)), ((), ())),
        preferred_element_type=jnp.float32)


_TILE = 256
_NT = 24


def _onehot(pos_ref, c_ref, off):
    slot = jax.lax.broadcasted_iota(jnp.int32, (_TILE, _S), 0) + off
    return (pos_ref[0] == slot.astype(jnp.float32)) & (c_ref[0] > 0.0)


def _gather_kernel(eot_ref, soff_ref, act_ref, pos_ref, c_ref, x_ref, xg_ref):
    m = pl.program_id(0)

    @pl.when(act_ref[m] == 1)
    def _():
        ohb = _onehot(pos_ref, c_ref, soff_ref[m]).astype(jnp.bfloat16)
        xg_ref[...] = jax.lax.dot_general(
            ohb, x_ref[...].astype(jnp.bfloat16), (((1,), (0,)), ((), ())),
            preferred_element_type=jnp.float32)

    @pl.when(act_ref[m] == 0)
    def _():
        xg_ref[...] = jnp.zeros((_TILE, _D), jnp.float32)


def _ffn_kernel(eot_ref, soff_ref, act_ref, xg_ref, w1_ref, b1_ref, w2_ref,
                b2_ref, o_ref):
    m = pl.program_id(0)

    @pl.when(act_ref[m] == 1)
    def _():
        h = _mm(xg_ref[...], w1_ref[0]) + b1_ref[0, 0]
        h = 0.5 * h * (1.0 + jax.lax.erf(h * (2.0 ** -0.5)))
        o_ref[...] = _mm(h, w2_ref[0]) + b2_ref[0, 0]

    @pl.when(act_ref[m] == 0)
    def _():
        o_ref[...] = jnp.zeros((_TILE, _D), jnp.float32)


def _scatter_kernel(eot_ref, soff_ref, act_ref, pos_ref, c_ref, gv_ref,
                    fo_ref, y_ref, o_ref):
    m = pl.program_id(0)

    @pl.when(m == 0)
    def _():
        o_ref[...] = y_ref[...]

    @pl.when(act_ref[m] == 1)
    def _():
        oh = _onehot(pos_ref, c_ref, soff_ref[m])
        ohw = (oh.astype(jnp.float32) * gv_ref[0]).astype(jnp.bfloat16)
        o_ref[...] += jax.lax.dot_general(
            ohw, fo_ref[...].astype(jnp.bfloat16), (((0,), (0,)), ((), ())),
            preferred_element_type=jnp.float32)


def kernel(x, mask, ln1_scale, ln1_bias, ln2_scale, ln2_bias,
           Wq, bq, Wk, bk, Wv, bv, Wo, bo, Wr, br, We1, be1, We2, be2):
    f32 = jnp.float32
    xf = x.reshape(_S, _D)
    mask2 = mask.reshape(1, _S)
    Wqkv = jnp.concatenate([Wq, Wk, Wv], axis=1)
    bqkv = jnp.concatenate([bq, bk, bv]).reshape(1, 3 * _D)
    l1s, l1b = ln1_scale.reshape(1, _D), ln1_bias.reshape(1, _D)
    l2s, l2b = ln2_scale.reshape(1, _D), ln2_bias.reshape(1, _D)

    qkv = pl.pallas_call(
        _qkv_kernel,
        grid=(_S // _BM,),
        in_specs=[
            pl.BlockSpec((_BM, _D), lambda i: (i, 0)),
            pl.BlockSpec((_D, 3 * _D), lambda i: (0, 0)),
            pl.BlockSpec((1, 3 * _D), lambda i: (0, 0)),
            pl.BlockSpec((1, _D), lambda i: (0, 0)),
            pl.BlockSpec((1, _D), lambda i: (0, 0)),
        ],
        out_specs=pl.BlockSpec((_BM, 3 * _D), lambda i: (i, 0)),
        out_shape=jax.ShapeDtypeStruct((_S, 3 * _D), f32),
    )(xf, Wqkv, bqkv, l1s, l1b)

    qh = qkv[:, :_D].reshape(_S, _H, _HD).transpose(1, 0, 2)
    kh = qkv[:, _D:2 * _D].reshape(_S, _H, _HD).transpose(1, 0, 2)
    vh = qkv[:, 2 * _D:].reshape(_S, _H, _HD).transpose(1, 0, 2)

    attn_w, attn_oh = pl.pallas_call(
        _attn_kernel,
        grid=(_H, _S // _BQ),
        in_specs=[
            pl.BlockSpec((1, _BQ, _HD), lambda h, q: (h, q, 0)),
            pl.BlockSpec((1, _S, _HD), lambda h, q: (h, 0, 0)),
            pl.BlockSpec((1, _S, _HD), lambda h, q: (h, 0, 0)),
            pl.BlockSpec((1, _S), lambda h, q: (0, 0)),
        ],
        out_specs=[
            pl.BlockSpec((1, 1, _BQ, _S), lambda h, q: (0, h, q, 0)),
            pl.BlockSpec((1, _BQ, _HD), lambda h, q: (h, q, 0)),
        ],
        out_shape=[
            jax.ShapeDtypeStruct((1, _H, _S, _S), f32),
            jax.ShapeDtypeStruct((_H, _S, _HD), f32),
        ],
    )(qh, kh, vh, mask2)

    attn_o = attn_oh.transpose(1, 0, 2).reshape(_S, _D)

    y = pl.pallas_call(
        _proj_kernel,
        grid=(_S // _BM,),
        in_specs=[
            pl.BlockSpec((_BM, _D), lambda i: (i, 0)),
            pl.BlockSpec((_D, _D), lambda i: (0, 0)),
            pl.BlockSpec((1, _D), lambda i: (0, 0)),
            pl.BlockSpec((_BM, _D), lambda i: (i, 0)),
        ],
        out_specs=pl.BlockSpec((_BM, _D), lambda i: (i, 0)),
        out_shape=jax.ShapeDtypeStruct((_S, _D), f32),
    )(attn_o, Wo, bo.reshape(1, _D), xf)

    normed2, probs_pad, gate, pos = pl.pallas_call(
        _router_kernel,
        out_shape=[
            jax.ShapeDtypeStruct((_S, _D), f32),
            jax.ShapeDtypeStruct((_S, _E), f32),
            jax.ShapeDtypeStruct((_S, _E), f32),
            jax.ShapeDtypeStruct((_S, _E), f32),
        ],
    )(y, l2s, l2b, Wr, br.reshape(1, _E))

    i32 = jnp.int32
    counts = (pos[_S - 1, :] + (gate[_S - 1, :] > 0)).astype(i32)
    nt_e = (counts + _TILE - 1) // _TILE
    ends = jnp.cumsum(nt_e)
    base = ends - nt_e
    mi = jnp.arange(_NT, dtype=i32)
    eot_raw = jnp.sum((mi[:, None] >= ends[None, :]).astype(i32), axis=1)
    active = (eot_raw < _E).astype(i32)
    eot = jnp.minimum(eot_raw, _E - 1)
    soff = (mi - base[eot]) * _TILE

    pos_t3 = pos.T.reshape(_E, 1, _S)
    c_t3 = (gate > 0).astype(f32).T.reshape(_E, 1, _S)
    gate_t3 = gate.T.reshape(_E, 1, _S)

    xg = pl.pallas_call(
        _gather_kernel,
        grid_spec=pltpu.PrefetchScalarGridSpec(
            num_scalar_prefetch=3,
            grid=(_NT,),
            in_specs=[
                pl.BlockSpec((1, 1, _S), lambda m, e, s, a: (e[m], 0, 0)),
                pl.BlockSpec((1, 1, _S), lambda m, e, s, a: (e[m], 0, 0)),
                pl.BlockSpec((_S, _D), lambda m, e, s, a: (0, 0)),
            ],
            out_specs=pl.BlockSpec((_TILE, _D), lambda m, e, s, a: (m, 0)),
        ),
        out_shape=jax.ShapeDtypeStruct((_NT * _TILE, _D), f32),
    )(eot, soff, active, pos_t3, c_t3, normed2)

    ffn = pl.pallas_call(
        _ffn_kernel,
        grid_spec=pltpu.PrefetchScalarGridSpec(
            num_scalar_prefetch=3,
            grid=(_NT,),
            in_specs=[
                pl.BlockSpec((_TILE, _D), lambda m, e, s, a: (m, 0)),
                pl.BlockSpec((1, _D, _DFF), lambda m, e, s, a: (e[m], 0, 0)),
                pl.BlockSpec((1, 1, _DFF), lambda m, e, s, a: (e[m], 0, 0)),
                pl.BlockSpec((1, _DFF, _D), lambda m, e, s, a: (e[m], 0, 0)),
                pl.BlockSpec((1, 1, _D), lambda m, e, s, a: (e[m], 0, 0)),
            ],
            out_specs=pl.BlockSpec((_TILE, _D), lambda m, e, s, a: (m, 0)),
        ),
        out_shape=jax.ShapeDtypeStruct((_NT * _TILE, _D), f32),
    )(eot, soff, active, xg, We1.astype(jnp.bfloat16),
      be1.reshape(_E, 1, _DFF), We2.astype(jnp.bfloat16),
      be2.reshape(_E, 1, _D))

    out = pl.pallas_call(
        _scatter_kernel,
        grid_spec=pltpu.PrefetchScalarGridSpec(
            num_scalar_prefetch=3,
            grid=(_NT,),
            in_specs=[
                pl.BlockSpec((1, 1, _S), lambda m, e, s, a: (e[m], 0, 0)),
                pl.BlockSpec((1, 1, _S), lambda m, e, s, a: (e[m], 0, 0)),
                pl.BlockSpec((1, 1, _S), lambda m, e, s, a: (e[m], 0, 0)),
                pl.BlockSpec((_TILE, _D), lambda m, e, s, a: (m, 0)),
                pl.BlockSpec((_S, _D), lambda m, e, s, a: (0, 0)),
            ],
            out_specs=pl.BlockSpec((_S, _D), lambda m, e, s, a: (0, 0)),
        ),
        out_shape=jax.ShapeDtypeStruct((_S, _D), f32),
    )(eot, soff, active, pos_t3, c_t3, gate_t3, ffn, y)

    return (out.reshape(1, _S, _D), attn_w,
            probs_pad[:, :_K].reshape(1, _S, _K))


# speedup vs baseline: 1.0330x; 1.0330x over previous
import jax
import jax.numpy as jnp
from jax.experimental import pallas as pl
from jax.experimental.pallas import tpu as pltpu

_S, _D, _H, _HD, _E, _K, _DFF = 2048, 1024, 16, 64, 8, 2, 4096
_BM = 256
_BQ = 256


def _ln_rows(x, scale, bias):
    m = jnp.mean(x, axis=-1, keepdims=True)
    xc = x - m
    v = jnp.mean(xc * xc, axis=-1, keepdims=True)
    return xc * jax.lax.rsqrt(v + 1e-5) * scale + bias


def _mm(a, b):
    return jax.lax.dot_general(
        a.astype(jnp.bfloat16), b.astype(jnp.bfloat16),
        (((1,), (0,)), ((), ())), preferred_element_type=jnp.float32)


def _mmT(a, b):
    return jax.lax.dot_general(
        a.astype(jnp.bfloat16), b.astype(jnp.bfloat16),
        (((1,), (1,)), ((), ())), preferred_element_type=jnp.float32)


def _qkv_kernel(x_ref, w_ref, b_ref, s_ref, t_ref, o_ref):
    xn = _ln_rows(x_ref[...], s_ref[...], t_ref[...])
    o_ref[...] = _mm(xn, w_ref[...]) + b_ref[...]


def _attn_kernel(q_ref, k_ref, v_ref, m_ref, w_ref, o_ref):
    s = _mmT(q_ref[0], k_ref[0]) * (_HD ** -0.5)
    s = jnp.where(m_ref[...] == 0.0, -1e30, s)
    mx = jnp.max(s, axis=-1, keepdims=True)
    p = jnp.exp(s - mx)
    p = p / jnp.sum(p, axis=-1, keepdims=True)
    w_ref[...] = p.reshape(1, 1, _BQ, _S)
    o_ref[...] = _mm(p, v_ref[0]).reshape(1, _BQ, _HD)


def _proj_kernel(a_ref, w_ref, b_ref, x_ref, o_ref):
    o_ref[...] = x_ref[...] + _mm(a_ref[...], w_ref[...]) + b_ref[...]


def _router_kernel(y_ref, s_ref, b_ref, wr_ref, br_ref, n_ref, p_ref, g_ref,
                   q_ref):
    yn = _ln_rows(y_ref[...], s_ref[...], b_ref[...])
    n_ref[...] = yn
    logits = _mm(yn, wr_ref[...]) + br_ref[...]
    iota = jax.lax.broadcasted_iota(jnp.int32, (_S, _E), 1)
    m1 = jnp.max(logits, axis=-1, keepdims=True)
    i1 = jnp.min(jnp.where(logits == m1, iota, _E), axis=-1, keepdims=True)
    l2 = jnp.where(iota == i1, -jnp.inf, logits)
    m2 = jnp.max(l2, axis=-1, keepdims=True)
    i2 = jnp.min(jnp.where(l2 == m2, iota, _E), axis=-1, keepdims=True)
    p1 = 1.0 / (1.0 + jnp.exp(m2 - m1))
    p2 = 1.0 - p1
    p_ref[...] = jnp.where(iota == 0, p1, jnp.where(iota == 1, p2, 0.0))
    g = (jnp.where(iota == i1, p1, 0.0)
         + jnp.where(iota == i2, p2, 0.0))
    g_ref[...] = g
    c = (g > 0.0).astype(jnp.bfloat16)
    ti = jax.lax.broadcasted_iota(jnp.int32, (_S, _S), 0)
    tj = jax.lax.broadcasted_iota(jnp.int32, (_S, _S), 1)
    lstrict = (tj < ti).astype(jnp.bfloat16)
    q_ref[...] = jax.lax.dot_general(
        lstrict, c, (((1,), (0,)), ((), ())),
        preferred_element_type=jnp.float32)


_TILE = 256
_NT = 24


def _onehot(pos_ref, c_ref, off):
    slot = jax.lax.broadcasted_iota(jnp.int32, (_TILE, _S), 0) + off
    return (pos_ref[0] == slot.astype(jnp.float32)) & (c_ref[0] > 0.0)


def _ffn_kernel(eot_ref, soff_ref, act_ref, pos_ref, c_ref, x_ref, w1_ref,
                b1_ref, w2_ref, b2_ref, o_ref):
    m = pl.program_id(0)

    @pl.when(act_ref[m] == 1)
    def _():
        ohb = _onehot(pos_ref, c_ref, soff_ref[m]).astype(jnp.bfloat16)
        xg = jax.lax.dot_general(
            ohb, x_ref[...].astype(jnp.bfloat16), (((1,), (0,)), ((), ())),
            preferred_element_type=jnp.float32)
        h = _mm(xg, w1_ref[0]) + b1_ref[0, 0]
        h = 0.5 * h * (1.0 + jax.lax.erf(h * (2.0 ** -0.5)))
        o_ref[...] = _mm(h, w2_ref[0]) + b2_ref[0, 0]

    @pl.when(act_ref[m] == 0)
    def _():
        o_ref[...] = jnp.zeros((_TILE, _D), jnp.float32)


def _scatter_kernel(eot_ref, soff_ref, act_ref, pos_ref, c_ref, gv_ref,
                    fo_ref, y_ref, o_ref):
    m = pl.program_id(0)

    @pl.when(m == 0)
    def _():
        o_ref[...] = y_ref[...]

    @pl.when(act_ref[m] == 1)
    def _():
        oh = _onehot(pos_ref, c_ref, soff_ref[m])
        ohw = (oh.astype(jnp.float32) * gv_ref[0]).astype(jnp.bfloat16)
        o_ref[...] += jax.lax.dot_general(
            ohw, fo_ref[...].astype(jnp.bfloat16), (((0,), (0,)), ((), ())),
            preferred_element_type=jnp.float32)


def kernel(x, mask, ln1_scale, ln1_bias, ln2_scale, ln2_bias,
           Wq, bq, Wk, bk, Wv, bv, Wo, bo, Wr, br, We1, be1, We2, be2):
    f32 = jnp.float32
    xf = x.reshape(_S, _D)
    mask2 = mask.reshape(1, _S)
    Wqkv = jnp.concatenate([Wq, Wk, Wv], axis=1)
    bqkv = jnp.concatenate([bq, bk, bv]).reshape(1, 3 * _D)
    l1s, l1b = ln1_scale.reshape(1, _D), ln1_bias.reshape(1, _D)
    l2s, l2b = ln2_scale.reshape(1, _D), ln2_bias.reshape(1, _D)

    qkv = pl.pallas_call(
        _qkv_kernel,
        grid=(_S // _BM,),
        in_specs=[
            pl.BlockSpec((_BM, _D), lambda i: (i, 0)),
            pl.BlockSpec((_D, 3 * _D), lambda i: (0, 0)),
            pl.BlockSpec((1, 3 * _D), lambda i: (0, 0)),
            pl.BlockSpec((1, _D), lambda i: (0, 0)),
            pl.BlockSpec((1, _D), lambda i: (0, 0)),
        ],
        out_specs=pl.BlockSpec((_BM, 3 * _D), lambda i: (i, 0)),
        out_shape=jax.ShapeDtypeStruct((_S, 3 * _D), f32),
    )(xf, Wqkv, bqkv, l1s, l1b)

    qh = qkv[:, :_D].reshape(_S, _H, _HD).transpose(1, 0, 2)
    kh = qkv[:, _D:2 * _D].reshape(_S, _H, _HD).transpose(1, 0, 2)
    vh = qkv[:, 2 * _D:].reshape(_S, _H, _HD).transpose(1, 0, 2)

    attn_w, attn_oh = pl.pallas_call(
        _attn_kernel,
        grid=(_H, _S // _BQ),
        in_specs=[
            pl.BlockSpec((1, _BQ, _HD), lambda h, q: (h, q, 0)),
            pl.BlockSpec((1, _S, _HD), lambda h, q: (h, 0, 0)),
            pl.BlockSpec((1, _S, _HD), lambda h, q: (h, 0, 0)),
            pl.BlockSpec((1, _S), lambda h, q: (0, 0)),
        ],
        out_specs=[
            pl.BlockSpec((1, 1, _BQ, _S), lambda h, q: (0, h, q, 0)),
            pl.BlockSpec((1, _BQ, _HD), lambda h, q: (h, q, 0)),
        ],
        out_shape=[
            jax.ShapeDtypeStruct((1, _H, _S, _S), f32),
            jax.ShapeDtypeStruct((_H, _S, _HD), f32),
        ],
    )(qh, kh, vh, mask2)

    attn_o = attn_oh.transpose(1, 0, 2).reshape(_S, _D)

    y = pl.pallas_call(
        _proj_kernel,
        grid=(_S // _BM,),
        in_specs=[
            pl.BlockSpec((_BM, _D), lambda i: (i, 0)),
            pl.BlockSpec((_D, _D), lambda i: (0, 0)),
            pl.BlockSpec((1, _D), lambda i: (0, 0)),
            pl.BlockSpec((_BM, _D), lambda i: (i, 0)),
        ],
        out_specs=pl.BlockSpec((_BM, _D), lambda i: (i, 0)),
        out_shape=jax.ShapeDtypeStruct((_S, _D), f32),
    )(attn_o, Wo, bo.reshape(1, _D), xf)

    normed2, probs_pad, gate, pos = pl.pallas_call(
        _router_kernel,
        out_shape=[
            jax.ShapeDtypeStruct((_S, _D), f32),
            jax.ShapeDtypeStruct((_S, _E), f32),
            jax.ShapeDtypeStruct((_S, _E), f32),
            jax.ShapeDtypeStruct((_S, _E), f32),
        ],
    )(y, l2s, l2b, Wr, br.reshape(1, _E))

    i32 = jnp.int32
    counts = (pos[_S - 1, :] + (gate[_S - 1, :] > 0)).astype(i32)
    nt_e = (counts + _TILE - 1) // _TILE
    ends = jnp.cumsum(nt_e)
    base = ends - nt_e
    mi = jnp.arange(_NT, dtype=i32)
    eot_raw = jnp.sum((mi[:, None] >= ends[None, :]).astype(i32), axis=1)
    active = (eot_raw < _E).astype(i32)
    eot = jnp.minimum(eot_raw, _E - 1)
    soff = (mi - base[eot]) * _TILE

    pos_t3 = pos.T.reshape(_E, 1, _S)
    c_t3 = (gate > 0).astype(f32).T.reshape(_E, 1, _S)
    gate_t3 = gate.T.reshape(_E, 1, _S)

    ffn = pl.pallas_call(
        _ffn_kernel,
        grid_spec=pltpu.PrefetchScalarGridSpec(
            num_scalar_prefetch=3,
            grid=(_NT,),
            in_specs=[
                pl.BlockSpec((1, 1, _S), lambda m, e, s, a: (e[m], 0, 0)),
                pl.BlockSpec((1, 1, _S), lambda m, e, s, a: (e[m], 0, 0)),
                pl.BlockSpec((_S, _D), lambda m, e, s, a: (0, 0)),
                pl.BlockSpec((1, _D, _DFF), lambda m, e, s, a: (e[m], 0, 0)),
                pl.BlockSpec((1, 1, _DFF), lambda m, e, s, a: (e[m], 0, 0)),
                pl.BlockSpec((1, _DFF, _D), lambda m, e, s, a: (e[m], 0, 0)),
                pl.BlockSpec((1, 1, _D), lambda m, e, s, a: (e[m], 0, 0)),
            ],
            out_specs=pl.BlockSpec((_TILE, _D), lambda m, e, s, a: (m, 0)),
        ),
        out_shape=jax.ShapeDtypeStruct((_NT * _TILE, _D), f32),
    )(eot, soff, active, pos_t3, c_t3, normed2, We1.astype(jnp.bfloat16),
      be1.reshape(_E, 1, _DFF), We2.astype(jnp.bfloat16),
      be2.reshape(_E, 1, _D))

    out = pl.pallas_call(
        _scatter_kernel,
        grid_spec=pltpu.PrefetchScalarGridSpec(
            num_scalar_prefetch=3,
            grid=(_NT,),
            in_specs=[
                pl.BlockSpec((1, 1, _S), lambda m, e, s, a: (e[m], 0, 0)),
                pl.BlockSpec((1, 1, _S), lambda m, e, s, a: (e[m], 0, 0)),
                pl.BlockSpec((1, 1, _S), lambda m, e, s, a: (e[m], 0, 0)),
                pl.BlockSpec((_TILE, _D), lambda m, e, s, a: (m, 0)),
                pl.BlockSpec((_S, _D), lambda m, e, s, a: (0, 0)),
            ],
            out_specs=pl.BlockSpec((_S, _D), lambda m, e, s, a: (0, 0)),
        ),
        out_shape=jax.ShapeDtypeStruct((_S, _D), f32),
    )(eot, soff, active, pos_t3, c_t3, gate_t3, ffn, y)

    return (out.reshape(1, _S, _D), attn_w,
            probs_pad[:, :_K].reshape(1, _S, _K))


# speedup vs baseline: 1.2529x; 1.2128x over previous
import jax
import jax.numpy as jnp
from jax.experimental import pallas as pl
from jax.experimental.pallas import tpu as pltpu

_S, _D, _H, _HD, _E, _K, _DFF = 2048, 1024, 16, 64, 8, 2, 4096
_BM = 256
_BQ = 256


def _ln_rows(x, scale, bias):
    m = jnp.mean(x, axis=-1, keepdims=True)
    xc = x - m
    v = jnp.mean(xc * xc, axis=-1, keepdims=True)
    return xc * jax.lax.rsqrt(v + 1e-5) * scale + bias


def _mm(a, b):
    return jax.lax.dot_general(
        a.astype(jnp.bfloat16), b.astype(jnp.bfloat16),
        (((1,), (0,)), ((), ())), preferred_element_type=jnp.float32)


def _mmT(a, b):
    return jax.lax.dot_general(
        a.astype(jnp.bfloat16), b.astype(jnp.bfloat16),
        (((1,), (1,)), ((), ())), preferred_element_type=jnp.float32)


def _qkv_kernel(x_ref, w_ref, b_ref, s_ref, t_ref, o_ref):
    xn = _ln_rows(x_ref[...], s_ref[...], t_ref[...])
    o_ref[...] = _mm(xn, w_ref[...]) + b_ref[...]


def _attn_kernel(q_ref, k_ref, v_ref, m_ref, w_ref, o_ref):
    q2 = q_ref[...]
    k2 = k_ref[...]
    v2 = v_ref[...]
    outs = []
    for j in (0, 1):
        cols = slice(j * _HD, (j + 1) * _HD)
        s = _mmT(q2[:, cols], k2[:, cols]) * (_HD ** -0.5)
        s = jnp.where(m_ref[...] == 0.0, -1e30, s)
        mx = jnp.max(s, axis=-1, keepdims=True)
        p = jnp.exp(s - mx)
        p = p / jnp.sum(p, axis=-1, keepdims=True)
        w_ref[0, j] = p
        outs.append(_mm(p, v2[:, cols]))
    o_ref[...] = jnp.concatenate(outs, axis=1)


def _proj_kernel(a_ref, w_ref, b_ref, x_ref, o_ref):
    o_ref[...] = x_ref[...] + _mm(a_ref[...], w_ref[...]) + b_ref[...]


def _router_kernel(y_ref, s_ref, b_ref, wr_ref, br_ref, n_ref, p_ref, g_ref,
                   q_ref):
    yn = _ln_rows(y_ref[...], s_ref[...], b_ref[...])
    n_ref[...] = yn
    logits = _mm(yn, wr_ref[...]) + br_ref[...]
    iota = jax.lax.broadcasted_iota(jnp.int32, (_S, _E), 1)
    m1 = jnp.max(logits, axis=-1, keepdims=True)
    i1 = jnp.min(jnp.where(logits == m1, iota, _E), axis=-1, keepdims=True)
    l2 = jnp.where(iota == i1, -jnp.inf, logits)
    m2 = jnp.max(l2, axis=-1, keepdims=True)
    i2 = jnp.min(jnp.where(l2 == m2, iota, _E), axis=-1, keepdims=True)
    p1 = 1.0 / (1.0 + jnp.exp(m2 - m1))
    p2 = 1.0 - p1
    p_ref[...] = jnp.where(iota == 0, p1, jnp.where(iota == 1, p2, 0.0))
    g = (jnp.where(iota == i1, p1, 0.0)
         + jnp.where(iota == i2, p2, 0.0))
    g_ref[...] = g
    c = (g > 0.0).astype(jnp.bfloat16)
    ti = jax.lax.broadcasted_iota(jnp.int32, (_S, _S), 0)
    tj = jax.lax.broadcasted_iota(jnp.int32, (_S, _S), 1)
    lstrict = (tj < ti).astype(jnp.bfloat16)
    q_ref[...] = jax.lax.dot_general(
        lstrict, c, (((1,), (0,)), ((), ())),
        preferred_element_type=jnp.float32)


_TILE = 256
_NT = 24


def _onehot(pos_ref, c_ref, off):
    slot = jax.lax.broadcasted_iota(jnp.int32, (_TILE, _S), 0) + off
    return (pos_ref[0] == slot.astype(jnp.float32)) & (c_ref[0] > 0.0)


def _ffn_kernel(eot_ref, soff_ref, act_ref, pos_ref, c_ref, x_ref, w1_ref,
                b1_ref, w2_ref, b2_ref, o_ref):
    m = pl.program_id(0)

    @pl.when(act_ref[m] == 1)
    def _():
        ohb = _onehot(pos_ref, c_ref, soff_ref[m]).astype(jnp.bfloat16)
        xg = jax.lax.dot_general(
            ohb, x_ref[...].astype(jnp.bfloat16), (((1,), (0,)), ((), ())),
            preferred_element_type=jnp.float32)
        h = _mm(xg, w1_ref[0]) + b1_ref[0, 0]
        h = 0.5 * h * (1.0 + jax.lax.erf(h * (2.0 ** -0.5)))
        o_ref[...] = _mm(h, w2_ref[0]) + b2_ref[0, 0]

    @pl.when(act_ref[m] == 0)
    def _():
        o_ref[...] = jnp.zeros((_TILE, _D), jnp.float32)


def _scatter_kernel(eot_ref, soff_ref, act_ref, pos_ref, c_ref, gv_ref,
                    fo_ref, y_ref, o_ref):
    m = pl.program_id(0)

    @pl.when(m == 0)
    def _():
        o_ref[...] = y_ref[...]

    @pl.when(act_ref[m] == 1)
    def _():
        oh = _onehot(pos_ref, c_ref, soff_ref[m])
        ohw = (oh.astype(jnp.float32) * gv_ref[0]).astype(jnp.bfloat16)
        o_ref[...] += jax.lax.dot_general(
            ohw, fo_ref[...].astype(jnp.bfloat16), (((0,), (0,)), ((), ())),
            preferred_element_type=jnp.float32)


def kernel(x, mask, ln1_scale, ln1_bias, ln2_scale, ln2_bias,
           Wq, bq, Wk, bk, Wv, bv, Wo, bo, Wr, br, We1, be1, We2, be2):
    f32 = jnp.float32
    xf = x.reshape(_S, _D)
    mask2 = mask.reshape(1, _S)
    Wqkv = jnp.concatenate([Wq, Wk, Wv], axis=1)
    bqkv = jnp.concatenate([bq, bk, bv]).reshape(1, 3 * _D)
    l1s, l1b = ln1_scale.reshape(1, _D), ln1_bias.reshape(1, _D)
    l2s, l2b = ln2_scale.reshape(1, _D), ln2_bias.reshape(1, _D)

    qkv = pl.pallas_call(
        _qkv_kernel,
        grid=(_S // _BM,),
        in_specs=[
            pl.BlockSpec((_BM, _D), lambda i: (i, 0)),
            pl.BlockSpec((_D, 3 * _D), lambda i: (0, 0)),
            pl.BlockSpec((1, 3 * _D), lambda i: (0, 0)),
            pl.BlockSpec((1, _D), lambda i: (0, 0)),
            pl.BlockSpec((1, _D), lambda i: (0, 0)),
        ],
        out_specs=pl.BlockSpec((_BM, 3 * _D), lambda i: (i, 0)),
        out_shape=jax.ShapeDtypeStruct((_S, 3 * _D), f32),
    )(xf, Wqkv, bqkv, l1s, l1b)

    nhp = _D // (2 * _HD)
    attn_w, attn_o = pl.pallas_call(
        _attn_kernel,
        grid=(_H // 2, _S // _BQ),
        in_specs=[
            pl.BlockSpec((_BQ, 2 * _HD), lambda h, q: (q, h)),
            pl.BlockSpec((_S, 2 * _HD), lambda h, q: (0, nhp + h)),
            pl.BlockSpec((_S, 2 * _HD), lambda h, q: (0, 2 * nhp + h)),
            pl.BlockSpec((1, _S), lambda h, q: (0, 0)),
        ],
        out_specs=[
            pl.BlockSpec((1, 2, _BQ, _S), lambda h, q: (0, h, q, 0)),
            pl.BlockSpec((_BQ, 2 * _HD), lambda h, q: (q, h)),
        ],
        out_shape=[
            jax.ShapeDtypeStruct((1, _H, _S, _S), f32),
            jax.ShapeDtypeStruct((_S, _D), f32),
        ],
    )(qkv, qkv, qkv, mask2)

    y = pl.pallas_call(
        _proj_kernel,
        grid=(_S // _BM,),
        in_specs=[
            pl.BlockSpec((_BM, _D), lambda i: (i, 0)),
            pl.BlockSpec((_D, _D), lambda i: (0, 0)),
            pl.BlockSpec((1, _D), lambda i: (0, 0)),
            pl.BlockSpec((_BM, _D), lambda i: (i, 0)),
        ],
        out_specs=pl.BlockSpec((_BM, _D), lambda i: (i, 0)),
        out_shape=jax.ShapeDtypeStruct((_S, _D), f32),
    )(attn_o, Wo, bo.reshape(1, _D), xf)

    normed2, probs_pad, gate, pos = pl.pallas_call(
        _router_kernel,
        out_shape=[
            jax.ShapeDtypeStruct((_S, _D), f32),
            jax.ShapeDtypeStruct((_S, _E), f32),
            jax.ShapeDtypeStruct((_S, _E), f32),
            jax.ShapeDtypeStruct((_S, _E), f32),
        ],
    )(y, l2s, l2b, Wr, br.reshape(1, _E))

    i32 = jnp.int32
    counts = (pos[_S - 1, :] + (gate[_S - 1, :] > 0)).astype(i32)
    nt_e = (counts + _TILE - 1) // _TILE
    ends = jnp.cumsum(nt_e)
    base = ends - nt_e
    mi = jnp.arange(_NT, dtype=i32)
    eot_raw = jnp.sum((mi[:, None] >= ends[None, :]).astype(i32), axis=1)
    active = (eot_raw < _E).astype(i32)
    eot = jnp.minimum(eot_raw, _E - 1)
    soff = (mi - base[eot]) * _TILE

    pos_t3 = pos.T.reshape(_E, 1, _S)
    c_t3 = (gate > 0).astype(f32).T.reshape(_E, 1, _S)
    gate_t3 = gate.T.reshape(_E, 1, _S)

    ffn = pl.pallas_call(
        _ffn_kernel,
        grid_spec=pltpu.PrefetchScalarGridSpec(
            num_scalar_prefetch=3,
            grid=(_NT,),
            in_specs=[
                pl.BlockSpec((1, 1, _S), lambda m, e, s, a: (e[m], 0, 0)),
                pl.BlockSpec((1, 1, _S), lambda m, e, s, a: (e[m], 0, 0)),
                pl.BlockSpec((_S, _D), lambda m, e, s, a: (0, 0)),
                pl.BlockSpec((1, _D, _DFF), lambda m, e, s, a: (e[m], 0, 0)),
                pl.BlockSpec((1, 1, _DFF), lambda m, e, s, a: (e[m], 0, 0)),
                pl.BlockSpec((1, _DFF, _D), lambda m, e, s, a: (e[m], 0, 0)),
                pl.BlockSpec((1, 1, _D), lambda m, e, s, a: (e[m], 0, 0)),
            ],
            out_specs=pl.BlockSpec((_TILE, _D), lambda m, e, s, a: (m, 0)),
        ),
        out_shape=jax.ShapeDtypeStruct((_NT * _TILE, _D), f32),
    )(eot, soff, active, pos_t3, c_t3, normed2, We1.astype(jnp.bfloat16),
      be1.reshape(_E, 1, _DFF), We2.astype(jnp.bfloat16),
      be2.reshape(_E, 1, _D))

    out = pl.pallas_call(
        _scatter_kernel,
        grid_spec=pltpu.PrefetchScalarGridSpec(
            num_scalar_prefetch=3,
            grid=(_NT,),
            in_specs=[
                pl.BlockSpec((1, 1, _S), lambda m, e, s, a: (e[m], 0, 0)),
                pl.BlockSpec((1, 1, _S), lambda m, e, s, a: (e[m], 0, 0)),
                pl.BlockSpec((1, 1, _S), lambda m, e, s, a: (e[m], 0, 0)),
                pl.BlockSpec((_TILE, _D), lambda m, e, s, a: (m, 0)),
                pl.BlockSpec((_S, _D), lambda m, e, s, a: (0, 0)),
            ],
            out_specs=pl.BlockSpec((_S, _D), lambda m, e, s, a: (0, 0)),
        ),
        out_shape=jax.ShapeDtypeStruct((_S, _D), f32),
    )(eot, soff, active, pos_t3, c_t3, gate_t3, ffn, y)

    return (out.reshape(1, _S, _D), attn_w,
            probs_pad[:, :_K].reshape(1, _S, _K))
